# baseline (speedup 1.0000x reference)
import functools

import jax
import jax.numpy as jnp
from jax import lax
from jax.experimental import pallas as pl
from jax.experimental.pallas import tpu as pltpu
from jax.experimental.pallas import tpu_sc as plsc

_EMBED_DIM = 32

_info = plsc.get_sparse_core_info()
_NC, _NS = _info.num_cores, _info.num_subcores
_NW = _NC * _NS


def _make_index_prep(batch: int, hist: int):
    b_per_w = batch // _NW
    rows_per_w = b_per_w * hist
    n_rows = batch * hist
    mesh = plsc.VectorSubcoreMesh(core_axis_name="c", subcore_axis_name="s")

    @functools.partial(
        pl.kernel,
        mesh=mesh,
        compiler_params=pltpu.CompilerParams(
            use_tc_tiling_on_sc=False, needs_layout_passes=False
        ),
        out_type=(
            jax.ShapeDtypeStruct((n_rows,), jnp.int32),
            jax.ShapeDtypeStruct((n_rows,), jnp.int32),
        ),
        scratch_types=[
            pltpu.VMEM((b_per_w, hist), jnp.int32),
            pltpu.VMEM((rows_per_w,), jnp.int32),
            pltpu.VMEM((rows_per_w,), jnp.int32),
        ],
    )
    def prep_kernel(x_hbm, sup_out, sub_out, idx2d, sup_v, sub_v):
        wid = lax.axis_index("s") * _NC + lax.axis_index("c")
        pltpu.sync_copy(x_hbm.at[pl.ds(wid * b_per_w, b_per_w)], idx2d)
        lanes = lax.iota(jnp.int32, 16)

        def flat_body(j, carry):
            m = j * 16 + lanes
            h = m // b_per_w
            bl = m % b_per_w
            v = plsc.load_gather(idx2d, [bl, h])
            sup_v[pl.ds(j * 16, 16)] = v // 4
            sub_v[pl.ds(j * 16, 16)] = (v % 4) * _EMBED_DIM
            return carry

        lax.fori_loop(0, rows_per_w // 16, flat_body, 0)
        base = wid * rows_per_w
        pltpu.sync_copy(sup_v, sup_out.at[pl.ds(base, rows_per_w)])
        pltpu.sync_copy(sub_v, sub_out.at[pl.ds(base, rows_per_w)])

    return prep_kernel


def _make_gather(batch: int, hist: int, vocab4: int):
    b_per_w = batch // _NW
    rows_per_w = b_per_w * hist
    half = b_per_w // 2
    n_chunks = 2 * hist
    mesh = plsc.VectorSubcoreMesh(core_axis_name="c", subcore_axis_name="s")

    @functools.partial(
        pl.kernel,
        mesh=mesh,
        compiler_params=pltpu.CompilerParams(needs_layout_passes=False),
        out_type=jax.ShapeDtypeStruct((hist * _EMBED_DIM, batch), jnp.float32),
        scratch_types=[
            pltpu.VMEM((rows_per_w,), jnp.int32),
            pltpu.VMEM((rows_per_w,), jnp.int32),
            pltpu.VMEM((2, half, 128), jnp.float32),
            pltpu.VMEM((2, _EMBED_DIM, half), jnp.float32),
            pltpu.SemaphoreType.DMA,
            pltpu.SemaphoreType.DMA,
            pltpu.SemaphoreType.DMA,
            pltpu.SemaphoreType.DMA,
        ],
    )
    def gather_kernel(tbl_hbm, sup_hbm, sub_hbm, out_hbm, sup_v, sub_v,
                      super_v, trows, g0, g1, s0, s1):
        wid = lax.axis_index("s") * _NC + lax.axis_index("c")
        base = wid * rows_per_w
        colbase = wid * b_per_w
        pltpu.sync_copy(sup_hbm.at[pl.ds(base, rows_per_w)], sup_v)
        pltpu.sync_copy(sub_hbm.at[pl.ds(base, rows_per_w)], sub_v)
        lanes = lax.iota(jnp.int32, 16)
        gsem = (g0, g1)
        ssem = (s0, s1)

        pltpu.async_copy(tbl_hbm.at[sup_v.at[pl.ds(0, half)]], super_v.at[0], g0)

        def chunk_pair(c2, carry):
            for b in (0, 1):
                c = 2 * c2 + b
                pltpu.make_async_copy(
                    tbl_hbm.at[sup_v.at[pl.ds(0, half)]], super_v.at[b], gsem[b]
                ).wait()
                cn = jnp.minimum(c + 1, n_chunks - 1)
                pltpu.async_copy(
                    tbl_hbm.at[sup_v.at[pl.ds(cn * half, half)]],
                    super_v.at[1 - b],
                    gsem[1 - b],
                )

                @pl.when(c2 >= 1)
                def _wait_store():
                    pltpu.make_async_copy(
                        trows.at[b],
                        out_hbm.at[pl.ds(0, _EMBED_DIM), pl.ds(0, half)],
                        ssem[b],
                    ).wait()

                def e_body(e, car):
                    for g in range(half // 16):
                        rows = g * 16 + lanes
                        cols = sub_v[pl.ds(c * half + g * 16, 16)] + e
                        v = plsc.load_gather(super_v.at[b], [rows, cols])
                        trows[b, e, pl.ds(g * 16, 16)] = v
                    return car

                lax.fori_loop(0, _EMBED_DIM, e_body, 0)

                h = c // 2
                hb = c % 2
                pltpu.async_copy(
                    trows.at[b],
                    out_hbm.at[
                        pl.ds(h * _EMBED_DIM, _EMBED_DIM),
                        pl.ds(colbase + hb * half, half),
                    ],
                    ssem[b],
                )
            return carry

        lax.fori_loop(0, n_chunks // 2, chunk_pair, 0)

        pltpu.make_async_copy(
            tbl_hbm.at[sup_v.at[pl.ds(0, half)]], super_v.at[0], g0
        ).wait()
        pltpu.make_async_copy(
            trows.at[0], out_hbm.at[pl.ds(0, _EMBED_DIM), pl.ds(0, half)], s0
        ).wait()
        pltpu.make_async_copy(
            trows.at[1], out_hbm.at[pl.ds(0, _EMBED_DIM), pl.ds(0, half)], s1
        ).wait()

    return gather_kernel


def kernel(x, table):
    batch, hist = x.shape
    vocab, embed = table.shape
    sup, sub = _make_index_prep(batch, hist)(x.astype(jnp.int32))
    table4 = table.reshape(vocab // 4, 4 * embed)
    out2d = _make_gather(batch, hist, vocab // 4)(table4, sup, sub)
    return out2d.reshape(hist, embed, batch).transpose(2, 0, 1)


# speedup vs baseline: 1.0530x; 1.0530x over previous
import functools

import jax
import jax.numpy as jnp
from jax import lax
from jax.experimental import pallas as pl
from jax.experimental.pallas import tpu as pltpu
from jax.experimental.pallas import tpu_sc as plsc

_EMBED_DIM = 32

_info = plsc.get_sparse_core_info()
_NC, _NS = _info.num_cores, _info.num_subcores
_NW = _NC * _NS


def _make_relayout(vocab: int, embed: int):
    vb = 128
    n_blocks = (vocab + vb - 1) // vb
    iters = (n_blocks + _NW - 1) // _NW + 1
    pairs = (iters + 1) // 2
    blk_elems = vb * embed
    mesh = plsc.VectorSubcoreMesh(core_axis_name="c", subcore_axis_name="s")

    @functools.partial(
        pl.kernel,
        mesh=mesh,
        compiler_params=pltpu.CompilerParams(needs_layout_passes=False),
        out_type=jax.ShapeDtypeStruct((n_blocks * blk_elems,), jnp.float32),
        scratch_types=[
            pltpu.VMEM((embed, vb), jnp.float32),
            pltpu.VMEM((embed, vb), jnp.float32),
            pltpu.VMEM((blk_elems,), jnp.float32),
            pltpu.VMEM((blk_elems,), jnp.float32),
            pltpu.SemaphoreType.DMA,
            pltpu.SemaphoreType.DMA,
            pltpu.SemaphoreType.DMA,
            pltpu.SemaphoreType.DMA,
        ],
    )
    def relayout_kernel(tt_hbm, out_hbm, tblk0, tblk1, rowblk0, rowblk1,
                        g0, g1, s0, s1):
        wid = lax.axis_index("s") * _NC + lax.axis_index("c")
        lanes = lax.iota(jnp.int32, 16)
        lanes32 = lanes * embed
        tblk = (tblk0, tblk1)
        rowblk = (rowblk0, rowblk1)
        gsem = (g0, g1)
        ssem = (s0, s1)
        def vstart_of(i):
            blk = jnp.minimum(wid + i * _NW, n_blocks - 1)
            return pl.multiple_of(blk * vb, vb)

        pltpu.async_copy(
            tt_hbm.at[:, pl.ds(vstart_of(0), vb)], tblk[0], g0
        )

        def pair_body(p, carry):
            for b in (0, 1):
                i = 2 * p + b
                pltpu.make_async_copy(
                    tt_hbm.at[:, pl.ds(0, vb)], tblk[b], gsem[b]
                ).wait()
                pltpu.async_copy(
                    tt_hbm.at[:, pl.ds(vstart_of(i + 1), vb)],
                    tblk[1 - b],
                    gsem[1 - b],
                )
                @pl.when(p >= 1)
                def _wait_store():
                    pltpu.make_async_copy(
                        rowblk[b], out_hbm.at[pl.ds(0, blk_elems)], ssem[b]
                    ).wait()

                def e_body(e, car):
                    for g in range(vb // 16):
                        v = tblk[b][e, pl.ds(g * 16, 16)]
                        cols = (g * 16) * embed + lanes32 + e
                        plsc.store_scatter(rowblk[b], [cols], v)
                    return car

                lax.fori_loop(0, embed, e_body, 0)

                pltpu.async_copy(
                    rowblk[b],
                    out_hbm.at[pl.ds(vstart_of(i) * embed, blk_elems)],
                    ssem[b],
                )
            return carry

        lax.fori_loop(0, pairs, pair_body, 0)

        pltpu.make_async_copy(
            tt_hbm.at[:, pl.ds(0, vb)], tblk[0], g0
        ).wait()
        pltpu.make_async_copy(
            rowblk[0], out_hbm.at[pl.ds(0, blk_elems)], s0
        ).wait()
        pltpu.make_async_copy(
            rowblk[1], out_hbm.at[pl.ds(0, blk_elems)], s1
        ).wait()

    return relayout_kernel


def _make_gather(batch: int, hist: int, chunk_b: int):
    b_per_w = batch // _NW
    assert b_per_w % chunk_b == 0
    n_chunks = b_per_w // chunk_b
    chunk = chunk_b * hist
    rows_per_w = b_per_w * hist
    n_rows = batch * hist
    mesh = plsc.VectorSubcoreMesh(core_axis_name="c", subcore_axis_name="s")

    @functools.partial(
        pl.kernel,
        mesh=mesh,
        compiler_params=pltpu.CompilerParams(
            use_tc_tiling_on_sc=False, needs_layout_passes=False
        ),
        out_type=jax.ShapeDtypeStruct((n_rows, _EMBED_DIM), jnp.float32),
        scratch_types=[
            pltpu.VMEM((b_per_w, hist), jnp.int32),
            pltpu.VMEM((rows_per_w,), jnp.int32),
            pltpu.VMEM((2, chunk, _EMBED_DIM), jnp.float32),
            pltpu.SemaphoreType.DMA,
            pltpu.SemaphoreType.DMA,
            pltpu.SemaphoreType.DMA,
            pltpu.SemaphoreType.DMA,
        ],
    )
    def gather_kernel(table_hbm, x_hbm, out_hbm, idx2d, idx_v, rows_v,
                      g0, g1, s0, s1):
        wid = lax.axis_index("s") * _NC + lax.axis_index("c")
        base = wid * rows_per_w
        pltpu.sync_copy(x_hbm.at[pl.ds(wid * b_per_w, b_per_w)], idx2d)

        lanes = lax.iota(jnp.int32, 16)

        def flat_body(j, carry):
            m = j * 16 + lanes
            v = plsc.load_gather(idx2d, [m // hist, m % hist])
            idx_v[pl.ds(j * 16, 16)] = v
            return carry

        lax.fori_loop(0, rows_per_w // 16, flat_body, 0)

        gsem = (g0, g1)
        ssem = (s0, s1)
        gathers = [None, None]
        stores = [None, None]
        gathers[0] = pltpu.async_copy(
            table_hbm.at[idx_v.at[pl.ds(0, chunk)]], rows_v.at[0], g0
        )
        for i in range(n_chunks):
            b = i % 2
            nb = (i + 1) % 2
            if i + 1 < n_chunks:
                if stores[nb] is not None:
                    stores[nb].wait()
                gathers[nb] = pltpu.async_copy(
                    table_hbm.at[idx_v.at[pl.ds((i + 1) * chunk, chunk)]],
                    rows_v.at[nb],
                    gsem[nb],
                )
            gathers[b].wait()
            stores[b] = pltpu.async_copy(
                rows_v.at[b], out_hbm.at[pl.ds(base + i * chunk, chunk)], ssem[b]
            )
        stores[(n_chunks - 1) % 2].wait()
        if n_chunks >= 2:
            stores[(n_chunks - 2) % 2].wait()

    return gather_kernel


def kernel(x, table):
    batch, hist = x.shape
    vocab, embed = table.shape
    table_lin = _make_relayout(vocab, embed)(table.T)
    vocab_pad = table_lin.shape[0] // embed
    out = _make_gather(batch, hist, 64)(
        table_lin.reshape(vocab_pad, embed), x.astype(jnp.int32)
    )
    return out.reshape(batch, hist, _EMBED_DIM)
